# Initial kernel scaffold; baseline (speedup 1.0000x reference)
import jax
import jax.numpy as jnp
from jax.experimental import pallas as pl


def kernel(x, edge_attr, W_a, W_T, b_T, W_e, W_ee, prelu_a, edge_index):
    raise NotImplementedError("write your pallas kernel here")


# speedup vs baseline: 1.7692x; 1.7692x over previous
import functools

import jax
import jax.numpy as jnp
from jax import lax
from jax.experimental import pallas as pl
from jax.experimental.pallas import tpu as pltpu
from jax.experimental.pallas import tpu_sc as plsc

N_NODES = 10000
N_EDGES = 320000
V = 128
EF = 16
CH = 64
NC = 2
NS = 16
NW = NC * NS
C = 40
N_PAD = 10240
E_PAD = 327680
EDGES_PER_W = E_PAD // NW
CHUNKS = EDGES_PER_W // C
ROWS_PER_SUB = N_PAD // NS

NODE_BLK = 640
EDGE_BLK = 1024


def _node_proj_body(x_ref, wd0_ref, ws0_ref, wd1_ref, ws1_ref,
                    pd0_ref, ps0_ref, pd1_ref, ps1_ref):
    xb = x_ref[...]
    pd0_ref[...] = jnp.dot(xb, wd0_ref[...], preferred_element_type=jnp.float32)
    ps0_ref[...] = jnp.dot(xb, ws0_ref[...], preferred_element_type=jnp.float32)
    pd1_ref[...] = jnp.dot(xb, wd1_ref[...], preferred_element_type=jnp.float32)
    ps1_ref[...] = jnp.dot(xb, ws1_ref[...], preferred_element_type=jnp.float32)


def _node_proj(x, wd0, ws0, wd1, ws1):
    grid = N_PAD // NODE_BLK
    wspec = lambda cols: pl.BlockSpec((V, cols), lambda i: (0, 0))
    ospec = lambda cols: pl.BlockSpec((NODE_BLK, cols), lambda i: (i, 0))
    return pl.pallas_call(
        _node_proj_body,
        grid=(grid,),
        in_specs=[pl.BlockSpec((NODE_BLK, V), lambda i: (i, 0)),
                  wspec(2 * V), wspec(2 * V),
                  wspec(V), wspec(V)],
        out_specs=[ospec(2 * V), ospec(2 * V),
                   ospec(V), ospec(V)],
        out_shape=[jax.ShapeDtypeStruct((N_PAD, 2 * V), jnp.float32),
                   jax.ShapeDtypeStruct((N_PAD, 2 * V), jnp.float32),
                   jax.ShapeDtypeStruct((N_PAD, V), jnp.float32),
                   jax.ShapeDtypeStruct((N_PAD, V), jnp.float32)],
    )(x, wd0, ws0, wd1, ws1)


def _edge_proj_body(ea_ref, we0_ref, we1_ref, e0_ref, e1_ref):
    eb = ea_ref[...]
    e0_ref[...] = jnp.dot(eb, we0_ref[...], preferred_element_type=jnp.float32)
    e1_ref[...] = jnp.dot(eb, we1_ref[...], preferred_element_type=jnp.float32)


def _edge_proj(edge_attr, we0, we1):
    grid = E_PAD // EDGE_BLK
    return pl.pallas_call(
        _edge_proj_body,
        grid=(grid,),
        in_specs=[pl.BlockSpec((EDGE_BLK, EF), lambda i: (i, 0)),
                  pl.BlockSpec((EF, CH + CH + EF), lambda i: (0, 0)),
                  pl.BlockSpec((EF, V), lambda i: (0, 0))],
        out_specs=[pl.BlockSpec((EDGE_BLK, CH + CH + EF), lambda i: (i, 0)),
                   pl.BlockSpec((EDGE_BLK, V), lambda i: (i, 0))],
        out_shape=[jax.ShapeDtypeStruct((E_PAD, CH + CH + EF), jnp.float32),
                   jax.ShapeDtypeStruct((E_PAD, V), jnp.float32)],
    )(edge_attr, we0, we1)


def _make_edge_pass(with_ef):
    DT = 2 * V if with_ef else V
    DE = CH + CH + (EF if with_ef else 0)
    mesh = plsc.VectorSubcoreMesh(core_axis_name="c", subcore_axis_name="s",
                                  num_cores=NC, num_subcores=NS)
    out_type = [jax.ShapeDtypeStruct((NC, N_PAD, V), jnp.float32)]
    if with_ef:
        out_type.append(jax.ShapeDtypeStruct((E_PAD, EF), jnp.float32))

    scratch = [
        pltpu.VMEM((C,), jnp.int32),
        pltpu.VMEM((C,), jnp.int32),
        pltpu.VMEM((C, DT), jnp.float32),
        pltpu.VMEM((C, DT), jnp.float32),
        pltpu.VMEM((C, DE), jnp.float32),
        pltpu.VMEM((C, V), jnp.float32),
        pltpu.VMEM((16,), jnp.float32),
    ]
    if with_ef:
        scratch.append(pltpu.VMEM((C, EF), jnp.float32))
    scratch += [
        pltpu.VMEM_SHARED((N_PAD, V), jnp.float32),
        pltpu.SemaphoreType.DMA,
        pltpu.SemaphoreType.DMA,
    ]

    def body(pd_hbm, ps_hbm, eall_hbm, src_hbm, dst_hbm, zeros_hbm, pa_hbm,
             *rest):
        if with_ef:
            (acc_out, ef_out,
             src_idx, dst_idx, srows, drows, erows, wuw_v, pa_v, ef_v,
             acc_sh, sem0, sem1) = rest
        else:
            (acc_out,
             src_idx, dst_idx, srows, drows, erows, wuw_v, pa_v,
             acc_sh, sem0, sem1) = rest
            ef_out = ef_v = None
        c = lax.axis_index("c")
        s = lax.axis_index("s")
        rsl = pl.ds(s * ROWS_PER_SUB, ROWS_PER_SUB)
        pltpu.sync_copy(zeros_hbm.at[rsl], acc_sh.at[rsl])
        pltpu.sync_copy(pa_hbm, pa_v)
        plsc.subcore_barrier()
        a_vec = pa_v[...]

        ebase = c * (E_PAD // NC) + s * EDGES_PER_W

        def chunk_body(k, carry):
            eb = ebase + k * C
            esl = pl.ds(eb, C)
            pltpu.sync_copy(src_hbm.at[esl], src_idx)
            pltpu.sync_copy(dst_hbm.at[esl], dst_idx)
            cp_s = pltpu.async_copy(ps_hbm.at[src_idx], srows, sem0)
            cp_d = pltpu.async_copy(pd_hbm.at[dst_idx], drows, sem1)
            pltpu.sync_copy(eall_hbm.at[esl], erows)
            cp_s.wait()
            cp_d.wait()

            def edge_body(e, carry2):
                for j in range(CH // 16):
                    sl = pl.ds(16 * j, 16)
                    lv = drows[e, sl] + srows[e, sl] + erows[e, sl]
                    lv = jnp.where(lv >= 0.0, lv, a_vec * lv)
                    wv = jnp.exp(lv)
                    sl2 = pl.ds(CH + 16 * j, 16)
                    uv = drows[e, sl2] + srows[e, sl2] + erows[e, sl2]
                    wuw_v[e, sl] = wv * uv
                    wuw_v[e, sl2] = wv
                if with_ef:
                    sl3 = pl.ds(2 * CH, EF)
                    ef_v[e, pl.ds(0, EF)] = (drows[e, sl3] + srows[e, sl3]
                                             + erows[e, sl3])
                return carry2

            lax.fori_loop(0, C, edge_body, 0)

            pltpu.sync_copy(wuw_v, acc_sh.at[dst_idx], add=True)
            if with_ef:
                pltpu.sync_copy(ef_v, ef_out.at[esl])
            return carry

        lax.fori_loop(0, CHUNKS, chunk_body, 0)

        plsc.subcore_barrier()
        pltpu.sync_copy(acc_sh.at[rsl], acc_out.at[c, rsl])

    return pl.kernel(body, out_type=tuple(out_type), mesh=mesh,
                     scratch_types=tuple(scratch))


_edge_pass_cached = functools.cache(_make_edge_pass)


def _combine_body(a0_ref, a1_ref, b_ref, out_ref):
    a0 = a0_ref[0] + a0_ref[1]
    a1 = a1_ref[0] + a1_ref[1]
    b = b_ref[0]
    h0 = jnp.where(a0[:, CH:] > 0.0,
                   a0[:, :CH] / a0[:, CH:] + b[:CH][None, :], 0.0)
    h1 = jnp.where(a1[:, CH:] > 0.0,
                   a1[:, :CH] / a1[:, CH:] + b[CH:][None, :], 0.0)
    out_ref[...] = jnp.concatenate([h0, h1], axis=1)


def _combine(acc0, acc1, b2d):
    grid = N_PAD // NODE_BLK
    ispec = pl.BlockSpec((NC, NODE_BLK, V), lambda i: (0, i, 0))
    return pl.pallas_call(
        _combine_body,
        grid=(grid,),
        in_specs=[ispec, ispec,
                  pl.BlockSpec((1, V), lambda i: (0, 0))],
        out_specs=pl.BlockSpec((NODE_BLK, V), lambda i: (i, 0)),
        out_shape=jax.ShapeDtypeStruct((N_PAD, V), jnp.float32),
    )(acc0, acc1, b2d)


@jax.jit
def _impl(x, edge_attr, W_a, W_T, b_T, W_e, W_ee, prelu_a, edge_index):
    pad_e = E_PAD - N_EDGES
    src = jnp.concatenate([edge_index[0], jnp.zeros((pad_e,), jnp.int32)])
    dst = jnp.concatenate([edge_index[1],
                           jnp.full((pad_e,), N_PAD - 1, jnp.int32)])
    x_pad = jnp.concatenate(
        [x, jnp.zeros((N_PAD - N_NODES, V), jnp.float32)], axis=0)
    ea_pad = jnp.concatenate(
        [edge_attr, jnp.zeros((pad_e, EF), jnp.float32)], axis=0)
    zpad = jnp.zeros((V, 2 * V - CH - CH - EF), jnp.float32)
    wd0 = jnp.concatenate([W_a[0:V, 0:CH], W_T[0:V, 0:CH], W_e, zpad], axis=1)
    ws0 = jnp.concatenate([W_a[V + EF:, 0:CH], W_T[V + EF:, 0:CH], W_e, zpad],
                          axis=1)
    wd1 = jnp.concatenate([W_a[0:V, CH:], W_T[0:V, CH:]], axis=1)
    ws1 = jnp.concatenate([W_a[V + EF:, CH:], W_T[V + EF:, CH:]], axis=1)
    we0 = jnp.concatenate([W_a[V:V + EF, 0:CH], W_T[V:V + EF, 0:CH], W_ee],
                          axis=1)
    we1 = jnp.concatenate([W_a[V:V + EF, CH:], W_T[V:V + EF, CH:]], axis=1)

    pd0, ps0, pd1, ps1 = _node_proj(x_pad, wd0, ws0, wd1, ws1)
    eall0, eall1 = _edge_proj(ea_pad, we0, we1)

    zeros = jnp.zeros((N_PAD, V), jnp.float32)
    pa_vec = jnp.full((16,), prelu_a, jnp.float32)

    acc0, new_e = _edge_pass_cached(True)(pd0, ps0, eall0, src, dst,
                                          zeros, pa_vec)
    acc1 = _edge_pass_cached(False)(pd1, ps1, eall1, src, dst, zeros, pa_vec)
    if isinstance(acc1, (tuple, list)):
        acc1 = acc1[0]

    new_n = _combine(acc0, acc1, b_T.reshape(1, V))
    return new_n[:N_NODES], new_e[:N_EDGES]


def kernel(x, edge_attr, W_a, W_T, b_T, W_e, W_ee, prelu_a, edge_index):
    return _impl(x, edge_attr, W_a, W_T, b_T, W_e, W_ee, prelu_a, edge_index)


# speedup vs baseline: 2.4844x; 1.4042x over previous
import functools

import jax
import jax.numpy as jnp
from jax import lax
from jax.experimental import pallas as pl
from jax.experimental.pallas import tpu as pltpu
from jax.experimental.pallas import tpu_sc as plsc

N_NODES = 10000
N_EDGES = 320000
V = 128
EF = 16
CH = 64
NC = 2
NS = 16
NW = NC * NS
C = 32
N_PAD = 10240
E_PAD = 327680
EDGES_PER_W = E_PAD // NW
CHUNKS = EDGES_PER_W // C
ROWS_PER_SUB = N_PAD // NS

NODE_BLK = 640
EDGE_BLK = 1024
DE0 = CH + EF
DE1 = CH


def _pack_pair(lo, hi):
    lo16 = jax.lax.bitcast_convert_type(lo.astype(jnp.bfloat16), jnp.uint16)
    hi16 = jax.lax.bitcast_convert_type(hi.astype(jnp.bfloat16), jnp.uint16)
    w = lo16.astype(jnp.uint32) | (hi16.astype(jnp.uint32) << 16)
    return jax.lax.bitcast_convert_type(w, jnp.float32)


def _node_proj_body(x_ref, *refs):
    xb = x_ref[...]
    for i in range(4):
        lo = jnp.dot(xb, refs[2 * i][...], preferred_element_type=jnp.float32)
        hi = jnp.dot(xb, refs[2 * i + 1][...],
                     preferred_element_type=jnp.float32)
        refs[8 + i][...] = _pack_pair(lo, hi)


def _node_proj(x, ws):
    grid = N_PAD // NODE_BLK
    wspec = pl.BlockSpec((V, V), lambda i: (0, 0))
    ospec = pl.BlockSpec((NODE_BLK, V), lambda i: (i, 0))
    oshape = jax.ShapeDtypeStruct((N_PAD, V), jnp.float32)
    return pl.pallas_call(
        _node_proj_body,
        grid=(grid,),
        in_specs=[pl.BlockSpec((NODE_BLK, V), lambda i: (i, 0))] +
                 [wspec] * 8,
        out_specs=[ospec] * 4,
        out_shape=[oshape] * 4,
    )(x, *ws)


def _edge_proj_body(ea_ref, lo0_ref, hi0_ref, lo1_ref, hi1_ref,
                    e0_ref, e1_ref):
    eb = ea_ref[...]
    e0_ref[...] = _pack_pair(
        jnp.dot(eb, lo0_ref[...], preferred_element_type=jnp.float32),
        jnp.dot(eb, hi0_ref[...], preferred_element_type=jnp.float32))
    e1_ref[...] = _pack_pair(
        jnp.dot(eb, lo1_ref[...], preferred_element_type=jnp.float32),
        jnp.dot(eb, hi1_ref[...], preferred_element_type=jnp.float32))


def _edge_proj(edge_attr, lo0, hi0, lo1, hi1):
    grid = E_PAD // EDGE_BLK
    return pl.pallas_call(
        _edge_proj_body,
        grid=(grid,),
        in_specs=[pl.BlockSpec((EDGE_BLK, EF), lambda i: (i, 0)),
                  pl.BlockSpec((EF, DE0), lambda i: (0, 0)),
                  pl.BlockSpec((EF, DE0), lambda i: (0, 0)),
                  pl.BlockSpec((EF, DE1), lambda i: (0, 0)),
                  pl.BlockSpec((EF, DE1), lambda i: (0, 0))],
        out_specs=[pl.BlockSpec((EDGE_BLK, DE0), lambda i: (i, 0)),
                   pl.BlockSpec((EDGE_BLK, DE1), lambda i: (i, 0))],
        out_shape=[jax.ShapeDtypeStruct((E_PAD, DE0), jnp.float32),
                   jax.ShapeDtypeStruct((E_PAD, DE1), jnp.float32)],
    )(edge_attr, lo0, hi0, lo1, hi1)


def _make_edge_pass(with_ef):
    DE = DE0 if with_ef else DE1
    mesh = plsc.VectorSubcoreMesh(core_axis_name="c", subcore_axis_name="s",
                                  num_cores=NC, num_subcores=NS)
    out_type = [jax.ShapeDtypeStruct((NC, N_PAD, V), jnp.float32)]
    if with_ef:
        out_type.append(jax.ShapeDtypeStruct((E_PAD, EF), jnp.float32))

    scratch = [
        pltpu.VMEM((C,), jnp.int32),
        pltpu.VMEM((C,), jnp.int32),
        pltpu.VMEM((C,), jnp.int32),
        pltpu.VMEM((C,), jnp.int32),
        pltpu.VMEM((C, V), jnp.float32),
        pltpu.VMEM((C, V), jnp.float32),
        pltpu.VMEM((C, V), jnp.float32),
        pltpu.VMEM((C, V), jnp.float32),
        pltpu.VMEM((C, DE), jnp.float32),
        pltpu.VMEM((C, DE), jnp.float32),
        pltpu.VMEM((C, V), jnp.float32),
        pltpu.VMEM((16,), jnp.float32),
    ]
    if with_ef:
        scratch.append(pltpu.VMEM((C, EF), jnp.float32))
    scratch += [
        pltpu.VMEM_SHARED((N_PAD, V), jnp.float32),
        pltpu.SemaphoreType.DMA,
        pltpu.SemaphoreType.DMA,
        pltpu.SemaphoreType.DMA,
        pltpu.SemaphoreType.DMA,
        pltpu.SemaphoreType.DMA,
        pltpu.SemaphoreType.DMA,
    ]

    def body(td_hbm, ts_hbm, eall_hbm, src_hbm, dst_hbm, zeros_hbm, pa_hbm,
             *rest):
        if with_ef:
            (acc_out, ef_out,
             si0, si1, di0, di1, sr0, sr1, dr0, dr1, er0, er1, wuw_v, pa_v,
             ef_v, acc_sh, ss0, ss1, sd0, sd1, se0, se1) = rest
        else:
            (acc_out,
             si0, si1, di0, di1, sr0, sr1, dr0, dr1, er0, er1, wuw_v, pa_v,
             acc_sh, ss0, ss1, sd0, sd1, se0, se1) = rest
            ef_out = ef_v = None
        bufs = ((si0, di0, sr0, dr0, er0, ss0, sd0, se0),
                (si1, di1, sr1, dr1, er1, ss1, sd1, se1))
        c = lax.axis_index("c")
        s = lax.axis_index("s")
        rsl = pl.ds(s * ROWS_PER_SUB, ROWS_PER_SUB)
        pltpu.sync_copy(zeros_hbm.at[rsl], acc_sh.at[rsl])
        pltpu.sync_copy(pa_hbm, pa_v)
        plsc.subcore_barrier()
        a_vec = pa_v[...]

        ebase = c * (E_PAD // NC) + s * EDGES_PER_W

        def fetch(k, b):
            si, di, sr, dr, er, ss, sd, se = bufs[b]
            esl = pl.ds(ebase + k * C, C)
            pltpu.sync_copy(src_hbm.at[esl], si)
            pltpu.sync_copy(dst_hbm.at[esl], di)
            cps = pltpu.async_copy(ts_hbm.at[si], sr, ss)
            cpd = pltpu.async_copy(td_hbm.at[di], dr, sd)
            cpe = pltpu.async_copy(eall_hbm.at[esl], er, se)
            return cps, cpd, cpe

        def unpack16(rows, e, j):
            word = rows[e, pl.ds(16 * j, 16)]
            return plsc.unpack(plsc.bitcast(word, jnp.bfloat16),
                               format=plsc.PackFormat.INTERLEAVED)

        def half_step(k, b, cps):
            si, di, sr, dr, er, ss, sd, se = bufs[b]
            for cp in cps:
                cp.wait()

            def edge_body(e, carry2):
                for j in range(CH // 16):
                    sA, sT = unpack16(sr, e, j)
                    dA, dT = unpack16(dr, e, j)
                    eA, eT = unpack16(er, e, j)
                    lv = dA + sA + eA
                    lv = jnp.where(lv >= 0.0, lv, a_vec * lv)
                    wv = jnp.exp(lv)
                    uv = dT + sT + eT
                    wuw_v[e, pl.ds(16 * j, 16)] = wv * uv
                    wuw_v[e, pl.ds(CH + 16 * j, 16)] = wv
                if with_ef:
                    sPE, _ = unpack16(sr, e, 4)
                    dPE, _ = unpack16(dr, e, 4)
                    ePE, _ = unpack16(er, e, 4)
                    ef_v[e, pl.ds(0, EF)] = sPE + dPE + ePE
                return carry2

            lax.fori_loop(0, C, edge_body, 0)

            pltpu.sync_copy(wuw_v, acc_sh.at[di], add=True)
            if with_ef:
                pltpu.sync_copy(ef_v, ef_out.at[pl.ds(ebase + k * C, C)])

        def chunk_pair(k2, carry):
            k0 = 2 * k2
            cps0 = fetch(k0, 0)
            cps1 = fetch(k0 + 1, 1)
            half_step(k0, 0, cps0)
            half_step(k0 + 1, 1, cps1)
            return carry

        lax.fori_loop(0, CHUNKS // 2, chunk_pair, 0)

        plsc.subcore_barrier()
        pltpu.sync_copy(acc_sh.at[rsl], acc_out.at[c, rsl])

    return pl.kernel(body, out_type=tuple(out_type), mesh=mesh,
                     scratch_types=tuple(scratch),
                     compiler_params=pltpu.CompilerParams(
                         needs_layout_passes=False))


_edge_pass_cached = functools.cache(_make_edge_pass)


def _combine_body(a0_ref, a1_ref, b_ref, out_ref):
    a0 = a0_ref[0] + a0_ref[1]
    a1 = a1_ref[0] + a1_ref[1]
    b = b_ref[0]
    h0 = jnp.where(a0[:, CH:] > 0.0,
                   a0[:, :CH] / a0[:, CH:] + b[:CH][None, :], 0.0)
    h1 = jnp.where(a1[:, CH:] > 0.0,
                   a1[:, :CH] / a1[:, CH:] + b[CH:][None, :], 0.0)
    out_ref[...] = jnp.concatenate([h0, h1], axis=1)


def _combine(acc0, acc1, b2d):
    grid = N_PAD // NODE_BLK
    ispec = pl.BlockSpec((NC, NODE_BLK, V), lambda i: (0, i, 0))
    return pl.pallas_call(
        _combine_body,
        grid=(grid,),
        in_specs=[ispec, ispec,
                  pl.BlockSpec((1, V), lambda i: (0, 0))],
        out_specs=pl.BlockSpec((NODE_BLK, V), lambda i: (i, 0)),
        out_shape=jax.ShapeDtypeStruct((N_PAD, V), jnp.float32),
    )(acc0, acc1, b2d)


@jax.jit
def _impl(x, edge_attr, W_a, W_T, b_T, W_e, W_ee, prelu_a, edge_index):
    pad_e = E_PAD - N_EDGES
    src = jnp.concatenate([edge_index[0], jnp.zeros((pad_e,), jnp.int32)])
    dst = jnp.concatenate([edge_index[1],
                           jnp.full((pad_e,), N_PAD - 1, jnp.int32)])
    x_pad = jnp.concatenate(
        [x, jnp.zeros((N_PAD - N_NODES, V), jnp.float32)], axis=0)
    ea_pad = jnp.concatenate(
        [edge_attr, jnp.zeros((pad_e, EF), jnp.float32)], axis=0)
    zn48 = jnp.zeros((V, V - CH - EF), jnp.float32)
    zn64 = jnp.zeros((V, V - CH), jnp.float32)
    ws = [
        jnp.concatenate([W_a[0:V, 0:CH], W_e, zn48], axis=1),
        jnp.concatenate([W_T[0:V, 0:CH], zn64], axis=1),
        jnp.concatenate([W_a[V + EF:, 0:CH], W_e, zn48], axis=1),
        jnp.concatenate([W_T[V + EF:, 0:CH], zn64], axis=1),
        jnp.concatenate([W_a[0:V, CH:], zn64], axis=1),
        jnp.concatenate([W_T[0:V, CH:], zn64], axis=1),
        jnp.concatenate([W_a[V + EF:, CH:], zn64], axis=1),
        jnp.concatenate([W_T[V + EF:, CH:], zn64], axis=1),
    ]
    ze16 = jnp.zeros((EF, EF), jnp.float32)
    elo0 = jnp.concatenate([W_a[V:V + EF, 0:CH], W_ee], axis=1)
    ehi0 = jnp.concatenate([W_T[V:V + EF, 0:CH], ze16], axis=1)
    elo1 = W_a[V:V + EF, CH:]
    ehi1 = W_T[V:V + EF, CH:]

    pd0, ps0, pd1, ps1 = _node_proj(x_pad, ws)
    eall0, eall1 = _edge_proj(ea_pad, elo0, ehi0, elo1, ehi1)

    zeros = jnp.zeros((N_PAD, V), jnp.float32)
    pa_vec = jnp.full((16,), prelu_a, jnp.float32)

    acc0, new_e = _edge_pass_cached(True)(pd0, ps0, eall0, src, dst,
                                          zeros, pa_vec)
    acc1 = _edge_pass_cached(False)(pd1, ps1, eall1, src, dst,
                                    zeros, pa_vec)
    if isinstance(acc1, (tuple, list)):
        acc1 = acc1[0]

    new_n = _combine(acc0, acc1, b_T.reshape(1, V))
    return new_n[:N_NODES], new_e[:N_EDGES]


def kernel(x, edge_attr, W_a, W_T, b_T, W_e, W_ee, prelu_a, edge_index):
    return _impl(x, edge_attr, W_a, W_T, b_T, W_e, W_ee, prelu_a, edge_index)
